# Initial kernel scaffold; baseline (speedup 1.0000x reference)
import jax
import jax.numpy as jnp
from jax.experimental import pallas as pl


def kernel(x, wg, fc1, b1, fc2, b2):
    raise NotImplementedError("write your pallas kernel here")


# speedup vs baseline: 1.0197x; 1.0197x over previous
import functools

import jax
import jax.numpy as jnp
from jax.experimental import pallas as pl
from jax.experimental.pallas import tpu as pltpu

T = 2048
M = 2048
E = 8
H = 4096
C = 256
BH = 512
NH = H // BH


def _ffn_body(xe_ref, fc1_ref, b1_ref, fc2_ref, b2_ref, out_ref, acc_ref):
    nh = pl.program_id(1)
    h = jnp.dot(xe_ref[0], fc1_ref[0], preferred_element_type=jnp.float32)
    h = jnp.maximum(h + b1_ref[0], 0.0)
    contrib = jnp.dot(h, fc2_ref[0], preferred_element_type=jnp.float32)

    @pl.when(nh == 0)
    def _():
        acc_ref[...] = contrib

    @pl.when(nh != 0)
    def _():
        acc_ref[...] += contrib

    @pl.when(nh == NH - 1)
    def _():
        out_ref[0] = acc_ref[...] + b2_ref[0]


@jax.jit
def _ffn(ei, fc1, b1, fc2, b2):
    return pl.pallas_call(
        _ffn_body,
        grid=(E, NH),
        in_specs=[
            pl.BlockSpec((1, C, M), lambda e, nh: (e, 0, 0)),
            pl.BlockSpec((1, M, BH), lambda e, nh: (e, 0, nh)),
            pl.BlockSpec((1, 1, BH), lambda e, nh: (e, 0, nh)),
            pl.BlockSpec((1, BH, M), lambda e, nh: (e, nh, 0)),
            pl.BlockSpec((1, 1, M), lambda e, nh: (e, 0, 0)),
        ],
        out_specs=pl.BlockSpec((1, C, M), lambda e, nh: (e, 0, 0)),
        out_shape=jax.ShapeDtypeStruct((E, C, M), jnp.float32),
        scratch_shapes=[pltpu.VMEM((C, M), jnp.float32)],
        compiler_params=pltpu.CompilerParams(
            dimension_semantics=("arbitrary", "arbitrary"),
        ),
    )(ei, fc1, b1, fc2, b2)


def kernel(x, wg, fc1, b1, fc2, b2):
    T_, M_ = x.shape
    E_ = wg.shape[0]
    capacity = C

    logits = x @ wg.T
    indices1_s = jnp.argmax(logits, axis=1)
    mask1 = jax.nn.one_hot(indices1_s, E_, dtype=logits.dtype)
    gates = jax.nn.softmax(logits, axis=1)
    gates1_s = jnp.sum(gates * mask1, axis=1)
    locations = jnp.cumsum(mask1, axis=0) - mask1
    locations1_s = jnp.sum(locations * mask1, axis=1).astype(jnp.int32)

    valid = locations1_s < capacity
    pos = indices1_s.astype(jnp.int32) * capacity + locations1_s
    pos_scatter = jnp.where(valid, pos, E_ * capacity)
    dispatched = jnp.zeros((E_ * capacity, M_), dtype=x.dtype).at[pos_scatter].add(
        gates1_s[:, None] * x, mode='drop')

    ei = dispatched.reshape(E_, capacity, M_)
    expert_output = _ffn(ei, fc1, b1.reshape(E, 1, H), fc2,
                         b2.reshape(E, 1, M)).reshape(E_ * capacity, M_)

    pos_safe = jnp.where(valid, pos, 0)
    gathered = jnp.where(valid[:, None], expert_output[pos_safe], 0.0)
    combined = gates1_s[:, None] * gathered
    return combined


# speedup vs baseline: 1.0272x; 1.0073x over previous
import functools

import jax
import jax.numpy as jnp
from jax.experimental import pallas as pl
from jax.experimental.pallas import tpu as pltpu

T = 2048
M = 2048
E = 8
H = 4096
C = 256
BH = 512
NH = H // BH


def _ffn_body(xe_ref, fc1_ref, b1_ref, fc2_ref, b2_ref, out_ref, acc_ref):
    nh = pl.program_id(1)
    h = jnp.dot(xe_ref[0].astype(jnp.bfloat16), fc1_ref[0].astype(jnp.bfloat16),
                preferred_element_type=jnp.float32)
    h = jnp.maximum(h + b1_ref[0], 0.0)
    contrib = jnp.dot(h.astype(jnp.bfloat16), fc2_ref[0].astype(jnp.bfloat16),
                      preferred_element_type=jnp.float32)

    @pl.when(nh == 0)
    def _():
        acc_ref[...] = contrib

    @pl.when(nh != 0)
    def _():
        acc_ref[...] += contrib

    @pl.when(nh == NH - 1)
    def _():
        out_ref[0] = acc_ref[...] + b2_ref[0]


@jax.jit
def _ffn(ei, fc1, b1, fc2, b2):
    return pl.pallas_call(
        _ffn_body,
        grid=(E, NH),
        in_specs=[
            pl.BlockSpec((1, C, M), lambda e, nh: (e, 0, 0)),
            pl.BlockSpec((1, M, BH), lambda e, nh: (e, 0, nh)),
            pl.BlockSpec((1, 1, BH), lambda e, nh: (e, 0, nh)),
            pl.BlockSpec((1, BH, M), lambda e, nh: (e, nh, 0)),
            pl.BlockSpec((1, 1, M), lambda e, nh: (e, 0, 0)),
        ],
        out_specs=pl.BlockSpec((1, C, M), lambda e, nh: (e, 0, 0)),
        out_shape=jax.ShapeDtypeStruct((E, C, M), jnp.float32),
        scratch_shapes=[pltpu.VMEM((C, M), jnp.float32)],
        compiler_params=pltpu.CompilerParams(
            dimension_semantics=("arbitrary", "arbitrary"),
        ),
    )(ei, fc1, b1, fc2, b2)


def kernel(x, wg, fc1, b1, fc2, b2):
    T_, M_ = x.shape
    E_ = wg.shape[0]
    capacity = C

    logits = x @ wg.T
    indices1_s = jnp.argmax(logits, axis=1)
    mask1 = jax.nn.one_hot(indices1_s, E_, dtype=logits.dtype)
    gates = jax.nn.softmax(logits, axis=1)
    gates1_s = jnp.sum(gates * mask1, axis=1)
    locations = jnp.cumsum(mask1, axis=0) - mask1
    locations1_s = jnp.sum(locations * mask1, axis=1).astype(jnp.int32)

    valid = locations1_s < capacity
    pos = indices1_s.astype(jnp.int32) * capacity + locations1_s
    pos_scatter = jnp.where(valid, pos, E_ * capacity)
    dispatched = jnp.zeros((E_ * capacity, M_), dtype=x.dtype).at[pos_scatter].add(
        gates1_s[:, None] * x, mode='drop')

    ei = dispatched.reshape(E_, capacity, M_)
    expert_output = _ffn(ei, fc1, b1.reshape(E, 1, H), fc2,
                         b2.reshape(E, 1, M)).reshape(E_ * capacity, M_)

    pos_safe = jnp.where(valid, pos, 0)
    gathered = jnp.where(valid[:, None], expert_output[pos_safe], 0.0)
    combined = gates1_s[:, None] * gathered
    return combined


# speedup vs baseline: 1.0524x; 1.0245x over previous
import functools

import jax
import jax.numpy as jnp
from jax.experimental import pallas as pl
from jax.experimental.pallas import tpu as pltpu

T = 2048
M = 2048
E = 8
H = 4096
C = 256
BH = 1024
NH = H // BH


def _ffn_body(xe_ref, fc1_ref, b1_ref, fc2_ref, b2_ref, out_ref, acc_ref):
    nh = pl.program_id(1)
    h = jnp.dot(xe_ref[0].astype(jnp.bfloat16), fc1_ref[0].astype(jnp.bfloat16),
                preferred_element_type=jnp.float32)
    h = jnp.maximum(h + b1_ref[0], 0.0)
    contrib = jnp.dot(h.astype(jnp.bfloat16), fc2_ref[0].astype(jnp.bfloat16),
                      preferred_element_type=jnp.float32)

    @pl.when(nh == 0)
    def _():
        acc_ref[...] = contrib

    @pl.when(nh != 0)
    def _():
        acc_ref[...] += contrib

    @pl.when(nh == NH - 1)
    def _():
        out_ref[0] = acc_ref[...] + b2_ref[0]


@jax.jit
def _ffn(ei, fc1, b1, fc2, b2):
    return pl.pallas_call(
        _ffn_body,
        grid=(E, NH),
        in_specs=[
            pl.BlockSpec((1, C, M), lambda e, nh: (e, 0, 0)),
            pl.BlockSpec((1, M, BH), lambda e, nh: (e, 0, nh)),
            pl.BlockSpec((1, 1, BH), lambda e, nh: (e, 0, nh)),
            pl.BlockSpec((1, BH, M), lambda e, nh: (e, nh, 0)),
            pl.BlockSpec((1, 1, M), lambda e, nh: (e, 0, 0)),
        ],
        out_specs=pl.BlockSpec((1, C, M), lambda e, nh: (e, 0, 0)),
        out_shape=jax.ShapeDtypeStruct((E, C, M), jnp.float32),
        scratch_shapes=[pltpu.VMEM((C, M), jnp.float32)],
        compiler_params=pltpu.CompilerParams(
            dimension_semantics=("arbitrary", "arbitrary"),
        ),
    )(ei, fc1, b1, fc2, b2)


def kernel(x, wg, fc1, b1, fc2, b2):
    T_, M_ = x.shape
    E_ = wg.shape[0]
    capacity = C

    logits = x @ wg.T
    indices1_s = jnp.argmax(logits, axis=1)
    mask1 = jax.nn.one_hot(indices1_s, E_, dtype=logits.dtype)
    gates = jax.nn.softmax(logits, axis=1)
    gates1_s = jnp.sum(gates * mask1, axis=1)
    locations = jnp.cumsum(mask1, axis=0) - mask1
    locations1_s = jnp.sum(locations * mask1, axis=1).astype(jnp.int32)

    valid = locations1_s < capacity
    pos = indices1_s.astype(jnp.int32) * capacity + locations1_s
    pos_scatter = jnp.where(valid, pos, E_ * capacity)
    dispatched = jnp.zeros((E_ * capacity, M_), dtype=x.dtype).at[pos_scatter].add(
        gates1_s[:, None] * x, mode='drop')

    ei = dispatched.reshape(E_, capacity, M_)
    expert_output = _ffn(ei, fc1, b1.reshape(E, 1, H), fc2,
                         b2.reshape(E, 1, M)).reshape(E_ * capacity, M_)

    pos_safe = jnp.where(valid, pos, 0)
    gathered = jnp.where(valid[:, None], expert_output[pos_safe], 0.0)
    combined = gates1_s[:, None] * gathered
    return combined


# speedup vs baseline: 1.0669x; 1.0138x over previous
import functools

import jax
import jax.numpy as jnp
from jax.experimental import pallas as pl
from jax.experimental.pallas import tpu as pltpu

T = 2048
M = 2048
E = 8
H = 4096
C = 256
BH = 1024
NH = H // BH


def _moe_body(outinit_ref, x_ref, ssrc_ref, svalid_ref, sgate_ref,
              fc1_ref, b1_ref, fc2_ref, b2_ref, out_ref,
              xe_raw, xs_ref, acc_ref, stage_ref, sem_in, sem_out):
    del outinit_ref
    e = pl.program_id(0)
    nh = pl.program_id(1)

    def issue_gather(e1, b):
        def body(c, _):
            t = ssrc_ref[e1, c]
            pltpu.make_async_copy(
                x_ref.at[pl.ds(t, 1), :],
                xe_raw.at[b, pl.ds(c, 1), :],
                sem_in.at[b],
            ).start()
            return 0
        jax.lax.fori_loop(0, C, body, 0, unroll=False)

    @pl.when(nh == 0)
    def _():
        b = jax.lax.rem(e, 2)

        @pl.when(e == 0)
        def _():
            issue_gather(0, 0)

        @pl.when(e + 1 < E)
        def _():
            issue_gather(e + 1, jax.lax.rem(e + 1, 2))

        pltpu.make_async_copy(
            x_ref.at[pl.ds(0, C), :], xe_raw.at[b], sem_in.at[b]
        ).wait()
        xs_ref[...] = (xe_raw[b] * sgate_ref[0]).astype(jnp.bfloat16)

    h = jnp.dot(xs_ref[...], fc1_ref[0].astype(jnp.bfloat16),
                preferred_element_type=jnp.float32)
    h = jnp.maximum(h + b1_ref[0], 0.0)
    contrib = jnp.dot(h.astype(jnp.bfloat16), fc2_ref[0].astype(jnp.bfloat16),
                      preferred_element_type=jnp.float32)

    @pl.when(nh == 0)
    def _():
        acc_ref[...] = contrib

    @pl.when(nh != 0)
    def _():
        acc_ref[...] += contrib

    @pl.when(nh == NH - 1)
    def _():
        stage_ref[...] = (acc_ref[...] + b2_ref[0]) * sgate_ref[0]

        def sbody(c, cnt):
            t = ssrc_ref[e, c]
            v = svalid_ref[e, c]

            def do_start():
                pltpu.make_async_copy(
                    stage_ref.at[pl.ds(c, 1), :],
                    out_ref.at[pl.ds(t, 1), :],
                    sem_out,
                ).start()

            jax.lax.cond(v == 1, do_start, lambda: None)
            return cnt + v

        cnt = jax.lax.fori_loop(0, C, sbody, 0, unroll=False)

        def wbody(i, _):
            pltpu.make_async_copy(
                x_ref.at[pl.ds(0, 1), :],
                stage_ref.at[pl.ds(0, 1), :],
                sem_out,
            ).wait()
            return 0
        jax.lax.fori_loop(0, cnt, wbody, 0, unroll=False)


@jax.jit
def _moe(outinit, x, ssrc, svalid, sgate, fc1, b1, fc2, b2):
    return pl.pallas_call(
        _moe_body,
        grid=(E, NH),
        in_specs=[
            pl.BlockSpec(memory_space=pltpu.MemorySpace.HBM),
            pl.BlockSpec(memory_space=pltpu.MemorySpace.HBM),
            pl.BlockSpec(memory_space=pltpu.SMEM),
            pl.BlockSpec(memory_space=pltpu.SMEM),
            pl.BlockSpec((1, C, 1), lambda e, nh: (e, 0, 0)),
            pl.BlockSpec((1, M, BH), lambda e, nh: (e, 0, nh)),
            pl.BlockSpec((1, 1, BH), lambda e, nh: (e, 0, nh)),
            pl.BlockSpec((1, BH, M), lambda e, nh: (e, nh, 0)),
            pl.BlockSpec((1, 1, M), lambda e, nh: (e, 0, 0)),
        ],
        out_specs=pl.BlockSpec(memory_space=pltpu.MemorySpace.HBM),
        out_shape=jax.ShapeDtypeStruct((T, M), jnp.float32),
        scratch_shapes=[
            pltpu.VMEM((2, C, M), jnp.float32),
            pltpu.VMEM((C, M), jnp.bfloat16),
            pltpu.VMEM((C, M), jnp.float32),
            pltpu.VMEM((C, M), jnp.float32),
            pltpu.SemaphoreType.DMA((2,)),
            pltpu.SemaphoreType.DMA,
        ],
        input_output_aliases={0: 0},
        compiler_params=pltpu.CompilerParams(
            dimension_semantics=("arbitrary", "arbitrary"),
        ),
    )(outinit, x, ssrc, svalid, sgate, fc1, b1, fc2, b2)


def kernel(x, wg, fc1, b1, fc2, b2):
    T_, M_ = x.shape
    E_ = wg.shape[0]

    logits = x @ wg.T
    indices1_s = jnp.argmax(logits, axis=1)
    mask1 = jax.nn.one_hot(indices1_s, E_, dtype=logits.dtype)
    gates = jax.nn.softmax(logits, axis=1)
    gates1_s = jnp.sum(gates * mask1, axis=1)
    locations = jnp.cumsum(mask1, axis=0) - mask1
    locations1_s = jnp.sum(locations * mask1, axis=1).astype(jnp.int32)

    valid = locations1_s < C
    pos = indices1_s.astype(jnp.int32) * C + locations1_s
    pos_scatter = jnp.where(valid, pos, E_ * C)
    tok = jnp.arange(T_, dtype=jnp.int32)
    ssrc = jnp.zeros((E_ * C + 1,), jnp.int32).at[pos_scatter].set(tok)[:E_ * C]
    svalid = jnp.zeros((E_ * C + 1,), jnp.int32).at[pos_scatter].set(1)[:E_ * C]
    sgate = jnp.zeros((E_ * C + 1,), jnp.float32).at[pos_scatter].set(gates1_s)[:E_ * C]

    out = _moe(jnp.zeros((T_, M_), jnp.float32), x,
               ssrc.reshape(E_, C), svalid.reshape(E_, C),
               sgate.reshape(E_, C, 1),
               fc1, b1.reshape(E_, 1, H), fc2, b2.reshape(E_, 1, M))
    return out
